# baseline (speedup 1.0000x reference)
import functools

import jax
import jax.numpy as jnp
from jax import lax
from jax.experimental import pallas as pl
from jax.experimental.pallas import tpu as pltpu
from jax.experimental.pallas import tpu_sc as plsc

F32 = jnp.float32
I32 = jnp.int32

W_EDGES = 128
W_SC = 80
NUM_CORES = 2
NUM_SUBCORES = 16
HALF = 128


def _mesh():
    return plsc.VectorSubcoreMesh(
        core_axis_name="c", subcore_axis_name="s",
        num_cores=NUM_CORES, num_subcores=NUM_SUBCORES)


def _node_body(x_ref, wm1_ref, wk1_ref, wqt_ref, xw_ref, xb_ref, *, temp):
    xb = x_ref[...]
    xw = jnp.dot(xb, wm1_ref[...], preferred_element_type=F32)
    p = jnp.dot(wk1_ref[...], wqt_ref[...], preferred_element_type=F32)
    xw_ref[0] = xw[:, :HALF]
    xw_ref[1] = xw[:, HALF:]
    xbv = jnp.dot(xb, p, preferred_element_type=F32) * (1.0 / temp)
    xb_ref[...] = jnp.concatenate(
        [xbv, jnp.zeros((xbv.shape[0], HALF - 16), F32)], axis=1)


def _node_kernel(x, wm1, wk1, wqt, temp):
    n, d = x.shape
    blk = 1000
    grid = n // blk
    return pl.pallas_call(
        functools.partial(_node_body, temp=temp),
        grid=(grid,),
        in_specs=[
            pl.BlockSpec((blk, d), lambda i: (i, 0)),
            pl.BlockSpec((d, d), lambda i: (0, 0)),
            pl.BlockSpec((d, d), lambda i: (0, 0)),
            pl.BlockSpec((d, 16), lambda i: (0, 0)),
        ],
        out_specs=[
            pl.BlockSpec((2, blk, HALF), lambda i: (0, i, 0)),
            pl.BlockSpec((blk, HALF), lambda i: (i, 0)),
        ],
        out_shape=[
            jax.ShapeDtypeStruct((2, n, HALF), F32),
            jax.ShapeDtypeStruct((n, HALF), F32),
        ],
    )(x, wm1, wk1, wqt)


def _xbs_body(xb_hbm, src_hbm, out_hbm, *scr, e_pad):
    idxb = scr[0:4]
    rows = scr[4:8]
    cmpb = scr[8]
    linsem = scr[9:13]
    gsem = scr[13:17]
    wid = lax.axis_index("s") * NUM_CORES + lax.axis_index("c")
    per = e_pad // (NUM_CORES * NUM_SUBCORES)
    nwin = per // W_EDGES

    def lin_start(w, r):
        base = wid * per + w * W_EDGES
        pltpu.async_copy(src_hbm.at[pl.ds(base, W_EDGES)], idxb[r], linsem[r])

    def lin_wait(r):
        pltpu.make_async_copy(src_hbm.at[pl.ds(0, W_EDGES)], idxb[r],
                              linsem[r]).wait()

    def gath_start(r):
        pltpu.async_copy(xb_hbm.at[idxb[r]], rows[r], gsem[r])

    def gath_wait(r):
        pltpu.make_async_copy(xb_hbm.at[idxb[r]], rows[r], gsem[r]).wait()

    lin_start(0, 0)
    lin_start(1, 1)
    lin_start(2, 2)
    lin_wait(0)
    gath_start(0)
    lin_wait(1)
    gath_start(1)

    def body(i, carry):
        for j in range(4):
            w = 4 * i + j
            r = j

            @pl.when(w + 3 < nwin)
            def _():
                lin_start(w + 3, (j + 3) % 4)

            @pl.when(w + 2 < nwin)
            def _():
                lin_wait((j + 2) % 4)
                gath_start((j + 2) % 4)

            gath_wait(r)
            for e in range(W_EDGES):
                cmpb[e, pl.ds(0, 16)] = rows[r][e, pl.ds(0, 16)]
            base = wid * per + w * W_EDGES
            pltpu.sync_copy(cmpb, out_hbm.at[pl.ds(base, W_EDGES)])
        return carry

    lax.fori_loop(0, nwin // 4, body, 0)


def _xbs_kernel(xb, srcp, e_pad):
    kern = pl.kernel(
        functools.partial(_xbs_body, e_pad=e_pad),
        out_type=jax.ShapeDtypeStruct((e_pad, 16), F32),
        mesh=_mesh(),
        scratch_types=(
            [pltpu.VMEM((W_EDGES,), I32)] * 4
            + [pltpu.VMEM((W_EDGES, HALF), F32)] * 4
            + [pltpu.VMEM((W_EDGES, 16), F32)]
            + [pltpu.SemaphoreType.DMA] * 8
        ),
    )
    return kern(xb, srcp)


def _edge_body(eh_ref, eq_ref, xbs_ref, wm2_ref, wk2_ref, wqt_ref,
               ew_ref, s_ref, *, temp, blk, e_valid):
    i = pl.program_id(0)
    eids = i * blk + lax.broadcasted_iota(I32, (blk, 1), 0)
    mask = eids < e_valid
    eh = eh_ref[...]
    ew = jnp.where(mask, jnp.dot(eh, wm2_ref[...], preferred_element_type=F32),
                   0.0)
    ew_ref[0] = ew[:, :HALF]
    ew_ref[1] = ew[:, HALF:]
    c = jnp.dot(wk2_ref[...], wqt_ref[...], preferred_element_type=F32)
    eq = eq_ref[...]
    att = (jnp.sum(eq * xbs_ref[:, :16], axis=1, keepdims=True)
           + jnp.sum(eq * jnp.dot(eh, c, preferred_element_type=F32),
                     axis=1, keepdims=True) * (1.0 / temp))
    s_ref[...] = jnp.where(mask, jnp.exp(att), 0.0)


def _edge_kernel(eh, eq, xbs, wm2, wk2, wqt, temp, e_valid, e_pad):
    de = eh.shape[1]
    d = wm2.shape[1]
    blk = 2048
    grid = e_pad // blk
    pb = (e_valid - 1) // blk

    def clamped(i):
        return (jnp.minimum(i, pb), 0)

    return pl.pallas_call(
        functools.partial(_edge_body, temp=temp, blk=blk, e_valid=e_valid),
        grid=(grid,),
        in_specs=[
            pl.BlockSpec((blk, de), clamped),
            pl.BlockSpec((blk, de), clamped),
            pl.BlockSpec((blk, 16), lambda i: (i, 0)),
            pl.BlockSpec((de, d), lambda i: (0, 0)),
            pl.BlockSpec((de, d), lambda i: (0, 0)),
            pl.BlockSpec((d, 16), lambda i: (0, 0)),
        ],
        out_specs=[
            pl.BlockSpec((2, blk, HALF), lambda i: (0, i, 0)),
            pl.BlockSpec((blk, 1), lambda i: (i, 0)),
        ],
        out_shape=[
            jax.ShapeDtypeStruct((2, e_pad, HALF), F32),
            jax.ShapeDtypeStruct((e_pad, 1), F32),
        ],
    )(eh, eq, xbs, wm2, wk2, wqt)


def _scatter_body(xw2_hbm, ew2_hbm, src_hbm, dst_hbm, s_hbm, z2_hbm, z1_hbm,
                  agg_out, den_out, *scr, n, n_pad, e_pad):
    srcb = scr[0:4]
    dstb = scr[4:8]
    sb = scr[8:12]
    ewb = scr[12:14]
    gidx = scr[14:16]
    rows = scr[16:18]
    aggsp, densp = scr[18], scr[19]
    linsem = scr[20:24]
    ewsem = scr[24:26]
    gsem = scr[26:28]

    c = lax.axis_index("c")
    t = lax.axis_index("s")
    nrows = n_pad // NUM_SUBCORES
    pltpu.sync_copy(z2_hbm.at[pl.ds(t * nrows, nrows)],
                    aggsp.at[pl.ds(t * nrows, nrows)])

    @pl.when(t == 0)
    def _():
        pltpu.sync_copy(z1_hbm, densp)

    plsc.subcore_barrier()

    ept = e_pad // NUM_SUBCORES
    nwin = ept // W_SC

    def idx_start(w, r):
        base = t * ept + w * W_SC
        pltpu.async_copy(src_hbm.at[pl.ds(base, W_SC)], srcb[r], linsem[r])
        pltpu.async_copy(dst_hbm.at[pl.ds(base, W_SC)], dstb[r], linsem[r])
        pltpu.async_copy(s_hbm.at[pl.ds(base, W_SC)], sb[r], linsem[r])

    def idx_wait(r):
        z = pl.ds(0, W_SC)
        pltpu.make_async_copy(src_hbm.at[z], srcb[r], linsem[r]).wait()
        pltpu.make_async_copy(dst_hbm.at[z], dstb[r], linsem[r]).wait()
        pltpu.make_async_copy(s_hbm.at[z], sb[r], linsem[r]).wait()

    def ew_start(w, r):
        base = t * ept + w * W_SC
        pltpu.async_copy(ew2_hbm.at[pl.ds(c * e_pad + base, W_SC)],
                         ewb[r], ewsem[r])

    def ew_wait(r):
        pltpu.make_async_copy(ew2_hbm.at[pl.ds(0, W_SC)], ewb[r],
                              ewsem[r]).wait()

    def gidx_compute(r4, r):
        for kk in range(W_SC // 16):
            sl = pl.ds(kk * 16, 16)
            gidx[r][sl] = srcb[r4][sl] + c * n

    def gath_start(r):
        pltpu.async_copy(xw2_hbm.at[gidx[r]], rows[r], gsem[r])

    def gath_wait(r):
        pltpu.make_async_copy(xw2_hbm.at[gidx[r]], rows[r], gsem[r]).wait()

    def compute(r, r2, r4):
        rw, ew, sw = rows[r], ewb[r2], sb[r4]

        def gbody(g, carry2):
            sv16 = sw[pl.ds(g * 16, 16)]
            for j in range(16):
                e = g * 16 + j
                sv = sv16[j]
                for kk in range(HALF // 16):
                    sl = pl.ds(kk * 16, 16)
                    v = rw[e, sl] + ew[e, sl]
                    rw[e, sl] = jnp.maximum(v, v * 0.01) * sv
            return carry2

        lax.fori_loop(0, W_SC // 16, gbody, 0)

    def scat_sync(r, r2, r4):
        pltpu.sync_copy(rows[r], aggsp.at[dstb[r4]], add=True)

        @pl.when(c == 0)
        def _():
            pltpu.sync_copy(sb[r4], densp.at[dstb[r4]], add=True)

    idx_start(0, 0)
    idx_start(1, 1)
    ew_start(0, 0)
    ew_start(1, 1)
    idx_wait(0)
    gidx_compute(0, 0)
    gath_start(0)

    def body(i, carry):
        for j in range(4):
            w = 4 * i + j
            r4, r2 = j, j % 2

            @pl.when(w + 2 < nwin)
            def _():
                idx_start(w + 2, (j + 2) % 4)

            @pl.when(w + 1 < nwin)
            def _():
                idx_wait((j + 1) % 4)
                gidx_compute((j + 1) % 4, (j + 1) % 2)
                gath_start((j + 1) % 2)

            gath_wait(r2)
            ew_wait(r2)
            compute(r2, r2, r4)
            scat_sync(r2, r2, r4)

            @pl.when(w + 2 < nwin)
            def _():
                ew_start(w + 2, r2)
        return carry

    lax.fori_loop(0, nwin // 4, body, 0)
    plsc.subcore_barrier()
    pltpu.sync_copy(aggsp.at[pl.ds(t * nrows, nrows)],
                    agg_out.at[c, pl.ds(t * nrows, nrows)])

    @pl.when((t == 0) & (c == 0))
    def _():
        pltpu.sync_copy(densp, den_out)


def _scatter_kernel(xw2, ew2, srcp, dstp, s, n, n_pad, e_pad):
    z2 = jnp.zeros((n_pad, HALF), F32)
    z1 = jnp.zeros((n_pad,), F32)
    kern = pl.kernel(
        functools.partial(_scatter_body, n=n, n_pad=n_pad, e_pad=e_pad),
        out_type=(jax.ShapeDtypeStruct((2, n_pad, HALF), F32),
                  jax.ShapeDtypeStruct((n_pad,), F32)),
        mesh=_mesh(),
        scratch_types=(
            [pltpu.VMEM((W_SC,), I32)] * 4
            + [pltpu.VMEM((W_SC,), I32)] * 4
            + [pltpu.VMEM((W_SC,), F32)] * 4
            + [pltpu.VMEM((W_SC, HALF), F32)] * 2
            + [pltpu.VMEM((W_SC,), I32)] * 2
            + [pltpu.VMEM((W_SC, HALF), F32)] * 2
            + [pltpu.VMEM_SHARED((n_pad, HALF), F32),
               pltpu.VMEM_SHARED((n_pad,), F32)]
            + [pltpu.SemaphoreType.DMA] * 8
        ),
    )
    return kern(xw2, ew2, srcp, dstp, s, z2, z1)


def _final_body(agg_ref, den_ref, x_ref, h_ref):
    den = den_ref[...]
    r = jnp.where(den > 0, 1.0 / jnp.where(den > 0, den, 1.0), 0.0)
    h = jnp.concatenate([agg_ref[0] * r, agg_ref[1] * r], axis=1)
    h_ref[...] = h + x_ref[...]


def _final_kernel(agg, den, x):
    n, d = x.shape
    blk = 1000
    grid = n // blk
    return pl.pallas_call(
        _final_body,
        grid=(grid,),
        in_specs=[
            pl.BlockSpec((2, blk, HALF), lambda i: (0, i, 0)),
            pl.BlockSpec((blk, 1), lambda i: (i, 0)),
            pl.BlockSpec((blk, d), lambda i: (i, 0)),
        ],
        out_specs=pl.BlockSpec((blk, d), lambda i: (i, 0)),
        out_shape=jax.ShapeDtypeStruct((n, d), F32),
    )(agg, den, x)


def kernel(x, edge_index, edge_h, edge_qrh, Wm, Wq, Wk):
    n, d = x.shape
    e = edge_index.shape[1]
    de = edge_h.shape[1]
    temp = float(d) ** 0.5

    import math
    chunk = math.lcm(NUM_SUBCORES * W_SC * 4,
                     NUM_CORES * NUM_SUBCORES * W_EDGES * 4)
    e_pad = ((e + chunk - 1) // chunk) * chunk
    pad = e_pad - e

    src = jnp.pad(edge_index[0], (0, pad))
    dst = jnp.pad(edge_index[1], (0, pad))

    wm1, wm2 = Wm[:d], Wm[d:]
    wk1, wk2 = Wk[:d], Wk[d:]
    wqt = jnp.transpose(Wq)

    xw2, xb = _node_kernel(x, wm1, wk1, wqt, temp)
    xbs = _xbs_kernel(xb, src, e_pad)
    ew2, s = _edge_kernel(edge_h, edge_qrh, xbs, wm2, wk2, wqt, temp, e, e_pad)

    xw2f = jnp.reshape(xw2, (2 * n, HALF))
    ew2f = jnp.reshape(ew2, (2 * e_pad, HALF))
    sf = jnp.reshape(s, (e_pad,))

    n_pad = ((n + 16 * 8 - 1) // (16 * 8)) * (16 * 8)
    agg, den = _scatter_kernel(xw2f, ew2f, src, dst, sf, n, n_pad, e_pad)
    return _final_kernel(agg, jnp.reshape(den, (n_pad, 1)), x)


# speedup vs baseline: 1.0135x; 1.0135x over previous
import functools

import jax
import jax.numpy as jnp
from jax import lax
from jax.experimental import pallas as pl
from jax.experimental.pallas import tpu as pltpu
from jax.experimental.pallas import tpu_sc as plsc

F32 = jnp.float32
I32 = jnp.int32

W_EDGES = 128
W_SC = 80
NUM_CORES = 2
NUM_SUBCORES = 16
HALF = 128


def _mesh():
    return plsc.VectorSubcoreMesh(
        core_axis_name="c", subcore_axis_name="s",
        num_cores=NUM_CORES, num_subcores=NUM_SUBCORES)


def _node_body(x_ref, wm1_ref, wk1_ref, wqt_ref, xw_ref, xb_ref, *, temp):
    xb = x_ref[...]
    xw = jnp.dot(xb, wm1_ref[...], preferred_element_type=F32)
    p = jnp.dot(wk1_ref[...], wqt_ref[...], preferred_element_type=F32)
    xw_ref[0] = xw[:, :HALF]
    xw_ref[1] = xw[:, HALF:]
    xbv = jnp.dot(xb, p, preferred_element_type=F32) * (1.0 / temp)
    xb_ref[...] = jnp.concatenate(
        [xbv, jnp.zeros((xbv.shape[0], HALF - 16), F32)], axis=1)


def _node_kernel(x, wm1, wk1, wqt, temp):
    n, d = x.shape
    blk = 1000
    grid = n // blk
    return pl.pallas_call(
        functools.partial(_node_body, temp=temp),
        grid=(grid,),
        in_specs=[
            pl.BlockSpec((blk, d), lambda i: (i, 0)),
            pl.BlockSpec((d, d), lambda i: (0, 0)),
            pl.BlockSpec((d, d), lambda i: (0, 0)),
            pl.BlockSpec((d, 16), lambda i: (0, 0)),
        ],
        out_specs=[
            pl.BlockSpec((2, blk, HALF), lambda i: (0, i, 0)),
            pl.BlockSpec((blk, HALF), lambda i: (i, 0)),
        ],
        out_shape=[
            jax.ShapeDtypeStruct((2, n, HALF), F32),
            jax.ShapeDtypeStruct((n, HALF), F32),
        ],
    )(x, wm1, wk1, wqt)


def _xbs_body(xb_hbm, src_hbm, out_hbm, *scr, e_pad):
    idxb = scr[0:4]
    rows = scr[4:8]
    cmpb = scr[8]
    linsem = scr[9:13]
    gsem = scr[13:17]
    wid = lax.axis_index("s") * NUM_CORES + lax.axis_index("c")
    per = e_pad // (NUM_CORES * NUM_SUBCORES)
    nwin = per // W_EDGES

    def lin_start(w, r):
        base = wid * per + w * W_EDGES
        pltpu.async_copy(src_hbm.at[pl.ds(base, W_EDGES)], idxb[r], linsem[r])

    def lin_wait(r):
        pltpu.make_async_copy(src_hbm.at[pl.ds(0, W_EDGES)], idxb[r],
                              linsem[r]).wait()

    def gath_start(r):
        pltpu.async_copy(xb_hbm.at[idxb[r]], rows[r], gsem[r])

    def gath_wait(r):
        pltpu.make_async_copy(xb_hbm.at[idxb[r]], rows[r], gsem[r]).wait()

    lin_start(0, 0)
    lin_start(1, 1)
    lin_start(2, 2)
    lin_wait(0)
    gath_start(0)
    lin_wait(1)
    gath_start(1)

    def body(i, carry):
        for j in range(4):
            w = 4 * i + j
            r = j

            @pl.when(w + 3 < nwin)
            def _():
                lin_start(w + 3, (j + 3) % 4)

            @pl.when(w + 2 < nwin)
            def _():
                lin_wait((j + 2) % 4)
                gath_start((j + 2) % 4)

            gath_wait(r)
            for e in range(W_EDGES):
                cmpb[e, pl.ds(0, 16)] = rows[r][e, pl.ds(0, 16)]
            base = wid * per + w * W_EDGES
            pltpu.sync_copy(cmpb, out_hbm.at[pl.ds(base, W_EDGES)])
        return carry

    lax.fori_loop(0, nwin // 4, body, 0)


def _xbs_kernel(xb, srcp, e_pad):
    kern = pl.kernel(
        functools.partial(_xbs_body, e_pad=e_pad),
        out_type=jax.ShapeDtypeStruct((e_pad, 16), F32),
        mesh=_mesh(),
        scratch_types=(
            [pltpu.VMEM((W_EDGES,), I32)] * 4
            + [pltpu.VMEM((W_EDGES, HALF), F32)] * 4
            + [pltpu.VMEM((W_EDGES, 16), F32)]
            + [pltpu.SemaphoreType.DMA] * 8
        ),
    )
    return kern(xb, srcp)


def _edge_body(eh_ref, eq_ref, xbs_ref, wm2_ref, wk2_ref, wqt_ref,
               ew_ref, s_ref, *, temp, blk, e_valid):
    i = pl.program_id(0)
    eids = i * blk + lax.broadcasted_iota(I32, (blk, 1), 0)
    mask = eids < e_valid
    eh = eh_ref[...]
    ew = jnp.where(mask, jnp.dot(eh, wm2_ref[...], preferred_element_type=F32),
                   0.0)
    ew_ref[0] = ew[:, :HALF]
    ew_ref[1] = ew[:, HALF:]
    c = jnp.dot(wk2_ref[...], wqt_ref[...], preferred_element_type=F32)
    eq = eq_ref[...]
    att = (jnp.sum(eq * xbs_ref[:, :16], axis=1, keepdims=True)
           + jnp.sum(eq * jnp.dot(eh, c, preferred_element_type=F32),
                     axis=1, keepdims=True) * (1.0 / temp))
    s_ref[...] = jnp.where(mask, jnp.exp(att), 0.0)


def _edge_kernel(eh, eq, xbs, wm2, wk2, wqt, temp, e_valid, e_pad):
    de = eh.shape[1]
    d = wm2.shape[1]
    blk = 2048
    grid = e_pad // blk
    pb = (e_valid - 1) // blk

    def clamped(i):
        return (jnp.minimum(i, pb), 0)

    return pl.pallas_call(
        functools.partial(_edge_body, temp=temp, blk=blk, e_valid=e_valid),
        grid=(grid,),
        in_specs=[
            pl.BlockSpec((blk, de), clamped),
            pl.BlockSpec((blk, de), clamped),
            pl.BlockSpec((blk, 16), lambda i: (i, 0)),
            pl.BlockSpec((de, d), lambda i: (0, 0)),
            pl.BlockSpec((de, d), lambda i: (0, 0)),
            pl.BlockSpec((d, 16), lambda i: (0, 0)),
        ],
        out_specs=[
            pl.BlockSpec((2, blk, HALF), lambda i: (0, i, 0)),
            pl.BlockSpec((blk, 1), lambda i: (i, 0)),
        ],
        out_shape=[
            jax.ShapeDtypeStruct((2, e_pad, HALF), F32),
            jax.ShapeDtypeStruct((e_pad, 1), F32),
        ],
    )(eh, eq, xbs, wm2, wk2, wqt)


def _scatter_body(xw2_hbm, ew2_hbm, src_hbm, dst_hbm, s_hbm,
                  agg_out, den_out, *scr, n, n_pad, e_pad):
    srcb = scr[0:4]
    dstb = scr[4:8]
    sb = scr[8:12]
    ewb = scr[12:14]
    gidx = scr[14:16]
    rows = scr[16:18]
    aggsp, densp = scr[18], scr[19]
    linsem = scr[20:24]
    ewsem = scr[24:26]
    gsem = scr[26:28]

    c = lax.axis_index("c")
    t = lax.axis_index("s")
    nrows = n_pad // NUM_SUBCORES
    zv = jnp.zeros((16,), F32)

    def zrow(a, carry):
        for kk in range(HALF // 16):
            rows[0][a, pl.ds(kk * 16, 16)] = zv
        return carry

    lax.fori_loop(0, W_SC, zrow, 0)
    for kk in range(W_SC // 16):
        sb[0][pl.ds(kk * 16, 16)] = zv

    nfull = nrows // W_SC
    rem = nrows - nfull * W_SC

    def zcp(q, carry):
        pltpu.sync_copy(rows[0], aggsp.at[pl.ds(t * nrows + q * W_SC, W_SC)])
        return carry

    lax.fori_loop(0, nfull, zcp, 0)
    if rem:
        pltpu.sync_copy(rows[0].at[pl.ds(0, rem)],
                        aggsp.at[pl.ds(t * nrows + nfull * W_SC, rem)])

    dfull = nrows // W_SC
    drem = nrows - dfull * W_SC

    def zcd(q, carry):
        pltpu.sync_copy(sb[0], densp.at[pl.ds(t * nrows + q * W_SC, W_SC)])
        return carry

    lax.fori_loop(0, dfull, zcd, 0)
    if drem:
        pltpu.sync_copy(sb[0].at[pl.ds(0, drem)],
                        densp.at[pl.ds(t * nrows + dfull * W_SC, drem)])

    plsc.subcore_barrier()

    ept = e_pad // NUM_SUBCORES
    nwin = ept // W_SC

    def idx_start(w, r):
        base = t * ept + w * W_SC
        pltpu.async_copy(src_hbm.at[pl.ds(base, W_SC)], srcb[r], linsem[r])
        pltpu.async_copy(dst_hbm.at[pl.ds(base, W_SC)], dstb[r], linsem[r])
        pltpu.async_copy(s_hbm.at[pl.ds(base, W_SC)], sb[r], linsem[r])

    def idx_wait(r):
        z = pl.ds(0, W_SC)
        pltpu.make_async_copy(src_hbm.at[z], srcb[r], linsem[r]).wait()
        pltpu.make_async_copy(dst_hbm.at[z], dstb[r], linsem[r]).wait()
        pltpu.make_async_copy(s_hbm.at[z], sb[r], linsem[r]).wait()

    def ew_start(w, r):
        base = t * ept + w * W_SC
        pltpu.async_copy(ew2_hbm.at[pl.ds(c * e_pad + base, W_SC)],
                         ewb[r], ewsem[r])

    def ew_wait(r):
        pltpu.make_async_copy(ew2_hbm.at[pl.ds(0, W_SC)], ewb[r],
                              ewsem[r]).wait()

    def gidx_compute(r4, r):
        for kk in range(W_SC // 16):
            sl = pl.ds(kk * 16, 16)
            gidx[r][sl] = srcb[r4][sl] + c * n

    def gath_start(r):
        pltpu.async_copy(xw2_hbm.at[gidx[r]], rows[r], gsem[r])

    def gath_wait(r):
        pltpu.make_async_copy(xw2_hbm.at[gidx[r]], rows[r], gsem[r]).wait()

    def compute(r, r2, r4):
        rw, ew, sw = rows[r], ewb[r2], sb[r4]

        def gbody(g, carry2):
            sv16 = sw[pl.ds(g * 16, 16)]
            for j in range(16):
                e = g * 16 + j
                sv = sv16[j]
                for kk in range(HALF // 16):
                    sl = pl.ds(kk * 16, 16)
                    v = rw[e, sl] + ew[e, sl]
                    rw[e, sl] = jnp.maximum(v, v * 0.01) * sv
            return carry2

        lax.fori_loop(0, W_SC // 16, gbody, 0)

    def scat_sync(r, r2, r4):
        pltpu.sync_copy(rows[r], aggsp.at[dstb[r4]], add=True)

        @pl.when(c == 0)
        def _():
            pltpu.sync_copy(sb[r4], densp.at[dstb[r4]], add=True)

    idx_start(0, 0)
    idx_start(1, 1)
    ew_start(0, 0)
    ew_start(1, 1)
    idx_wait(0)
    gidx_compute(0, 0)
    gath_start(0)

    def body(i, carry):
        for j in range(4):
            w = 4 * i + j
            r4, r2 = j, j % 2

            @pl.when(w + 2 < nwin)
            def _():
                idx_start(w + 2, (j + 2) % 4)

            @pl.when(w + 1 < nwin)
            def _():
                idx_wait((j + 1) % 4)
                gidx_compute((j + 1) % 4, (j + 1) % 2)
                gath_start((j + 1) % 2)

            gath_wait(r2)
            ew_wait(r2)
            compute(r2, r2, r4)
            scat_sync(r2, r2, r4)

            @pl.when(w + 2 < nwin)
            def _():
                ew_start(w + 2, r2)
        return carry

    lax.fori_loop(0, nwin // 4, body, 0)
    plsc.subcore_barrier()
    pltpu.sync_copy(aggsp.at[pl.ds(t * nrows, nrows)],
                    agg_out.at[c, pl.ds(t * nrows, nrows)])

    @pl.when((t == 0) & (c == 0))
    def _():
        pltpu.sync_copy(densp, den_out)


def _scatter_kernel(xw2, ew2, srcp, dstp, s, n, n_pad, e_pad):
    kern = pl.kernel(
        functools.partial(_scatter_body, n=n, n_pad=n_pad, e_pad=e_pad),
        out_type=(jax.ShapeDtypeStruct((2, n_pad, HALF), F32),
                  jax.ShapeDtypeStruct((n_pad,), F32)),
        mesh=_mesh(),
        scratch_types=(
            [pltpu.VMEM((W_SC,), I32)] * 4
            + [pltpu.VMEM((W_SC,), I32)] * 4
            + [pltpu.VMEM((W_SC,), F32)] * 4
            + [pltpu.VMEM((W_SC, HALF), F32)] * 2
            + [pltpu.VMEM((W_SC,), I32)] * 2
            + [pltpu.VMEM((W_SC, HALF), F32)] * 2
            + [pltpu.VMEM_SHARED((n_pad, HALF), F32),
               pltpu.VMEM_SHARED((n_pad,), F32)]
            + [pltpu.SemaphoreType.DMA] * 8
        ),
    )
    return kern(xw2, ew2, srcp, dstp, s)


def _final_body(agg_ref, den_ref, x_ref, h_ref):
    den = den_ref[...]
    r = jnp.where(den > 0, 1.0 / jnp.where(den > 0, den, 1.0), 0.0)
    h = jnp.concatenate([agg_ref[0] * r, agg_ref[1] * r], axis=1)
    h_ref[...] = h + x_ref[...]


def _final_kernel(agg, den, x):
    n, d = x.shape
    blk = 1000
    grid = n // blk
    return pl.pallas_call(
        _final_body,
        grid=(grid,),
        in_specs=[
            pl.BlockSpec((2, blk, HALF), lambda i: (0, i, 0)),
            pl.BlockSpec((blk, 1), lambda i: (i, 0)),
            pl.BlockSpec((blk, d), lambda i: (i, 0)),
        ],
        out_specs=pl.BlockSpec((blk, d), lambda i: (i, 0)),
        out_shape=jax.ShapeDtypeStruct((n, d), F32),
    )(agg, den, x)


def kernel(x, edge_index, edge_h, edge_qrh, Wm, Wq, Wk):
    n, d = x.shape
    e = edge_index.shape[1]
    de = edge_h.shape[1]
    temp = float(d) ** 0.5

    import math
    chunk = math.lcm(NUM_SUBCORES * W_SC * 4,
                     NUM_CORES * NUM_SUBCORES * W_EDGES * 4)
    e_pad = ((e + chunk - 1) // chunk) * chunk
    pad = e_pad - e

    src = jnp.pad(edge_index[0], (0, pad))
    dst = jnp.pad(edge_index[1], (0, pad))

    wm1, wm2 = Wm[:d], Wm[d:]
    wk1, wk2 = Wk[:d], Wk[d:]
    wqt = jnp.transpose(Wq)

    xw2, xb = _node_kernel(x, wm1, wk1, wqt, temp)
    xbs = _xbs_kernel(xb, src, e_pad)
    ew2, s = _edge_kernel(edge_h, edge_qrh, xbs, wm2, wk2, wqt, temp, e, e_pad)

    xw2f = jnp.reshape(xw2, (2 * n, HALF))
    ew2f = jnp.reshape(ew2, (2 * e_pad, HALF))
    sf = jnp.reshape(s, (e_pad,))

    n_pad = ((n + 16 * 8 - 1) // (16 * 8)) * (16 * 8)
    agg, den = _scatter_kernel(xw2f, ew2f, src, dst, sf, n, n_pad, e_pad)
    return _final_kernel(agg, jnp.reshape(den, (n_pad, 1)), x)


# speedup vs baseline: 1.0195x; 1.0059x over previous
import functools

import jax
import jax.numpy as jnp
from jax import lax
from jax.experimental import pallas as pl
from jax.experimental.pallas import tpu as pltpu
from jax.experimental.pallas import tpu_sc as plsc

F32 = jnp.float32
I32 = jnp.int32

W_EDGES = 128
W_SC = 80
NUM_CORES = 2
NUM_SUBCORES = 16
HALF = 128


def _mesh():
    return plsc.VectorSubcoreMesh(
        core_axis_name="c", subcore_axis_name="s",
        num_cores=NUM_CORES, num_subcores=NUM_SUBCORES)


def _node_body(x_ref, wm1_ref, wk1_ref, wqt_ref, xw_ref, xb_ref, *, temp):
    xb = x_ref[...]
    xw = jnp.dot(xb, wm1_ref[...], preferred_element_type=F32)
    p = jnp.dot(wk1_ref[...], wqt_ref[...], preferred_element_type=F32)
    xw_ref[0] = xw[:, :HALF]
    xw_ref[1] = xw[:, HALF:]
    xbv = jnp.dot(xb, p, preferred_element_type=F32) * (1.0 / temp)
    xb_ref[...] = jnp.concatenate(
        [xbv, jnp.zeros((xbv.shape[0], HALF - 16), F32)], axis=1)


def _node_kernel(x, wm1, wk1, wqt, temp):
    n, d = x.shape
    blk = 1000
    grid = n // blk
    return pl.pallas_call(
        functools.partial(_node_body, temp=temp),
        grid=(grid,),
        in_specs=[
            pl.BlockSpec((blk, d), lambda i: (i, 0)),
            pl.BlockSpec((d, d), lambda i: (0, 0)),
            pl.BlockSpec((d, d), lambda i: (0, 0)),
            pl.BlockSpec((d, 16), lambda i: (0, 0)),
        ],
        out_specs=[
            pl.BlockSpec((2, blk, HALF), lambda i: (0, i, 0)),
            pl.BlockSpec((blk, HALF), lambda i: (i, 0)),
        ],
        out_shape=[
            jax.ShapeDtypeStruct((2, n, HALF), F32),
            jax.ShapeDtypeStruct((n, HALF), F32),
        ],
    )(x, wm1, wk1, wqt)


def _xbs_body(xb_hbm, src_hbm, out_hbm, *scr, e_pad):
    idxb = scr[0:4]
    rows = scr[4:8]
    cmpb = scr[8]
    linsem = scr[9:13]
    gsem = scr[13:17]
    wid = lax.axis_index("s") * NUM_CORES + lax.axis_index("c")
    per = e_pad // (NUM_CORES * NUM_SUBCORES)
    nwin = per // W_EDGES

    def lin_start(w, r):
        base = wid * per + w * W_EDGES
        pltpu.async_copy(src_hbm.at[pl.ds(base, W_EDGES)], idxb[r], linsem[r])

    def lin_wait(r):
        pltpu.make_async_copy(src_hbm.at[pl.ds(0, W_EDGES)], idxb[r],
                              linsem[r]).wait()

    def gath_start(r):
        pltpu.async_copy(xb_hbm.at[idxb[r]], rows[r], gsem[r])

    def gath_wait(r):
        pltpu.make_async_copy(xb_hbm.at[idxb[r]], rows[r], gsem[r]).wait()

    lin_start(0, 0)
    lin_start(1, 1)
    lin_start(2, 2)
    lin_wait(0)
    gath_start(0)
    lin_wait(1)
    gath_start(1)

    def body(i, carry):
        for j in range(4):
            w = 4 * i + j
            r = j

            @pl.when(w + 3 < nwin)
            def _():
                lin_start(w + 3, (j + 3) % 4)

            @pl.when(w + 2 < nwin)
            def _():
                lin_wait((j + 2) % 4)
                gath_start((j + 2) % 4)

            gath_wait(r)
            for e in range(W_EDGES):
                cmpb[e, pl.ds(0, 16)] = rows[r][e, pl.ds(0, 16)]
            base = wid * per + w * W_EDGES
            pltpu.sync_copy(cmpb, out_hbm.at[pl.ds(base, W_EDGES)])
        return carry

    lax.fori_loop(0, nwin // 4, body, 0)


def _xbs_kernel(xb, srcp, e_pad):
    kern = pl.kernel(
        functools.partial(_xbs_body, e_pad=e_pad),
        out_type=jax.ShapeDtypeStruct((e_pad, 16), F32),
        mesh=_mesh(),
        scratch_types=(
            [pltpu.VMEM((W_EDGES,), I32)] * 4
            + [pltpu.VMEM((W_EDGES, HALF), F32)] * 4
            + [pltpu.VMEM((W_EDGES, 16), F32)]
            + [pltpu.SemaphoreType.DMA] * 8
        ),
    )
    return kern(xb, srcp)


def _ew_body(eh_ref, wm2_ref, ew_ref, *, blk, e_valid):
    i = pl.program_id(0)
    eids = i * blk + lax.broadcasted_iota(I32, (blk, 1), 0)
    mask = eids < e_valid
    ew = jnp.where(mask, jnp.dot(eh_ref[...], wm2_ref[...],
                                 preferred_element_type=F32), 0.0)
    ew_ref[0] = ew[:, :HALF]
    ew_ref[1] = ew[:, HALF:]


def _ew_kernel(eh, wm2, e_valid, e_pad):
    de = eh.shape[1]
    d = wm2.shape[1]
    blk = 2048
    grid = e_pad // blk
    pb = (e_valid - 1) // blk
    return pl.pallas_call(
        functools.partial(_ew_body, blk=blk, e_valid=e_valid),
        grid=(grid,),
        in_specs=[
            pl.BlockSpec((blk, de), lambda i: (jnp.minimum(i, pb), 0)),
            pl.BlockSpec((de, d), lambda i: (0, 0)),
        ],
        out_specs=pl.BlockSpec((2, blk, HALF), lambda i: (0, i, 0)),
        out_shape=jax.ShapeDtypeStruct((2, e_pad, HALF), F32),
    )(eh, wm2)


def _s_body(eh_ref, eq_ref, xbs_ref, wk2_ref, wqt_ref, s_ref,
            *, temp, blk, e_valid):
    i = pl.program_id(0)
    eids = i * blk + lax.broadcasted_iota(I32, (blk, 1), 0)
    mask = eids < e_valid
    c = jnp.dot(wk2_ref[...], wqt_ref[...], preferred_element_type=F32)
    eq = eq_ref[...]
    att = (jnp.sum(eq * xbs_ref[:, :16], axis=1, keepdims=True)
           + jnp.sum(eq * jnp.dot(eh_ref[...], c,
                                  preferred_element_type=F32),
                     axis=1, keepdims=True) * (1.0 / temp))
    s_ref[...] = jnp.where(mask, jnp.exp(att), 0.0)


def _s_kernel(eh, eq, xbs, wk2, wqt, temp, e_valid, e_pad):
    de = eh.shape[1]
    d = wk2.shape[1]
    blk = 2048
    grid = e_pad // blk
    pb = (e_valid - 1) // blk

    def clamped(i):
        return (jnp.minimum(i, pb), 0)

    return pl.pallas_call(
        functools.partial(_s_body, temp=temp, blk=blk, e_valid=e_valid),
        grid=(grid,),
        in_specs=[
            pl.BlockSpec((blk, de), clamped),
            pl.BlockSpec((blk, de), clamped),
            pl.BlockSpec((blk, 16), lambda i: (i, 0)),
            pl.BlockSpec((de, d), lambda i: (0, 0)),
            pl.BlockSpec((d, 16), lambda i: (0, 0)),
        ],
        out_specs=pl.BlockSpec((blk, 1), lambda i: (i, 0)),
        out_shape=jax.ShapeDtypeStruct((e_pad, 1), F32),
    )(eh, eq, xbs, wk2, wqt)


def _scatter_body(xw2_hbm, ew2_hbm, src_hbm, dst_hbm, s_hbm,
                  agg_out, den_out, *scr, n, n_pad, e_pad):
    srcb = scr[0:4]
    dstb = scr[4:8]
    sb = scr[8:12]
    ewb = scr[12:14]
    gidx = scr[14:16]
    rows = scr[16:18]
    aggsp, densp = scr[18], scr[19]
    linsem = scr[20:24]
    ewsem = scr[24:26]
    gsem = scr[26:28]

    c = lax.axis_index("c")
    t = lax.axis_index("s")
    nrows = n_pad // NUM_SUBCORES
    zv = jnp.zeros((16,), F32)

    def zrow(a, carry):
        for kk in range(HALF // 16):
            rows[0][a, pl.ds(kk * 16, 16)] = zv
        return carry

    lax.fori_loop(0, W_SC, zrow, 0)
    for kk in range(W_SC // 16):
        sb[0][pl.ds(kk * 16, 16)] = zv

    nfull = nrows // W_SC
    rem = nrows - nfull * W_SC

    def zcp(q, carry):
        pltpu.sync_copy(rows[0], aggsp.at[pl.ds(t * nrows + q * W_SC, W_SC)])
        return carry

    lax.fori_loop(0, nfull, zcp, 0)
    if rem:
        pltpu.sync_copy(rows[0].at[pl.ds(0, rem)],
                        aggsp.at[pl.ds(t * nrows + nfull * W_SC, rem)])

    dfull = nrows // W_SC
    drem = nrows - dfull * W_SC

    def zcd(q, carry):
        pltpu.sync_copy(sb[0], densp.at[pl.ds(t * nrows + q * W_SC, W_SC)])
        return carry

    lax.fori_loop(0, dfull, zcd, 0)
    if drem:
        pltpu.sync_copy(sb[0].at[pl.ds(0, drem)],
                        densp.at[pl.ds(t * nrows + dfull * W_SC, drem)])

    plsc.subcore_barrier()

    ept = e_pad // NUM_SUBCORES
    nwin = ept // W_SC

    def idx_start(w, r):
        base = t * ept + w * W_SC
        pltpu.async_copy(src_hbm.at[pl.ds(base, W_SC)], srcb[r], linsem[r])
        pltpu.async_copy(dst_hbm.at[pl.ds(base, W_SC)], dstb[r], linsem[r])
        pltpu.async_copy(s_hbm.at[pl.ds(base, W_SC)], sb[r], linsem[r])

    def idx_wait(r):
        z = pl.ds(0, W_SC)
        pltpu.make_async_copy(src_hbm.at[z], srcb[r], linsem[r]).wait()
        pltpu.make_async_copy(dst_hbm.at[z], dstb[r], linsem[r]).wait()
        pltpu.make_async_copy(s_hbm.at[z], sb[r], linsem[r]).wait()

    def ew_start(w, r):
        base = t * ept + w * W_SC
        pltpu.async_copy(ew2_hbm.at[pl.ds(c * e_pad + base, W_SC)],
                         ewb[r], ewsem[r])

    def ew_wait(r):
        pltpu.make_async_copy(ew2_hbm.at[pl.ds(0, W_SC)], ewb[r],
                              ewsem[r]).wait()

    def gidx_compute(r4, r):
        for kk in range(W_SC // 16):
            sl = pl.ds(kk * 16, 16)
            gidx[r][sl] = srcb[r4][sl] + c * n

    def gath_start(r):
        pltpu.async_copy(xw2_hbm.at[gidx[r]], rows[r], gsem[r])

    def gath_wait(r):
        pltpu.make_async_copy(xw2_hbm.at[gidx[r]], rows[r], gsem[r]).wait()

    def compute(r, r2, r4):
        rw, ew, sw = rows[r], ewb[r2], sb[r4]

        def gbody(g, carry2):
            sv16 = sw[pl.ds(g * 16, 16)]
            for j in range(16):
                e = g * 16 + j
                sv = sv16[j]
                for kk in range(HALF // 16):
                    sl = pl.ds(kk * 16, 16)
                    v = rw[e, sl] + ew[e, sl]
                    rw[e, sl] = jnp.maximum(v, v * 0.01) * sv
            return carry2

        lax.fori_loop(0, W_SC // 16, gbody, 0)

    def scat_sync(r, r2, r4):
        pltpu.sync_copy(rows[r], aggsp.at[dstb[r4]], add=True)

        @pl.when(c == 0)
        def _():
            pltpu.sync_copy(sb[r4], densp.at[dstb[r4]], add=True)

    idx_start(0, 0)
    idx_start(1, 1)
    ew_start(0, 0)
    ew_start(1, 1)
    idx_wait(0)
    gidx_compute(0, 0)
    gath_start(0)

    def body(i, carry):
        for j in range(4):
            w = 4 * i + j
            r4, r2 = j, j % 2

            @pl.when(w + 2 < nwin)
            def _():
                idx_start(w + 2, (j + 2) % 4)

            @pl.when(w + 1 < nwin)
            def _():
                idx_wait((j + 1) % 4)
                gidx_compute((j + 1) % 4, (j + 1) % 2)
                gath_start((j + 1) % 2)

            gath_wait(r2)
            ew_wait(r2)
            compute(r2, r2, r4)
            scat_sync(r2, r2, r4)

            @pl.when(w + 2 < nwin)
            def _():
                ew_start(w + 2, r2)
        return carry

    lax.fori_loop(0, nwin // 4, body, 0)
    plsc.subcore_barrier()
    pltpu.sync_copy(aggsp.at[pl.ds(t * nrows, nrows)],
                    agg_out.at[c, pl.ds(t * nrows, nrows)])

    @pl.when((t == 0) & (c == 0))
    def _():
        pltpu.sync_copy(densp, den_out)


def _scatter_kernel(xw2, ew2, srcp, dstp, s, n, n_pad, e_pad):
    kern = pl.kernel(
        functools.partial(_scatter_body, n=n, n_pad=n_pad, e_pad=e_pad),
        out_type=(jax.ShapeDtypeStruct((2, n_pad, HALF), F32),
                  jax.ShapeDtypeStruct((n_pad,), F32)),
        mesh=_mesh(),
        scratch_types=(
            [pltpu.VMEM((W_SC,), I32)] * 4
            + [pltpu.VMEM((W_SC,), I32)] * 4
            + [pltpu.VMEM((W_SC,), F32)] * 4
            + [pltpu.VMEM((W_SC, HALF), F32)] * 2
            + [pltpu.VMEM((W_SC,), I32)] * 2
            + [pltpu.VMEM((W_SC, HALF), F32)] * 2
            + [pltpu.VMEM_SHARED((n_pad, HALF), F32),
               pltpu.VMEM_SHARED((n_pad,), F32)]
            + [pltpu.SemaphoreType.DMA] * 8
        ),
    )
    return kern(xw2, ew2, srcp, dstp, s)


def _final_body(agg_ref, den_ref, x_ref, h_ref):
    den = den_ref[...]
    r = jnp.where(den > 0, 1.0 / jnp.where(den > 0, den, 1.0), 0.0)
    h = jnp.concatenate([agg_ref[0] * r, agg_ref[1] * r], axis=1)
    h_ref[...] = h + x_ref[...]


def _final_kernel(agg, den, x):
    n, d = x.shape
    blk = 1000
    grid = n // blk
    return pl.pallas_call(
        _final_body,
        grid=(grid,),
        in_specs=[
            pl.BlockSpec((2, blk, HALF), lambda i: (0, i, 0)),
            pl.BlockSpec((blk, 1), lambda i: (i, 0)),
            pl.BlockSpec((blk, d), lambda i: (i, 0)),
        ],
        out_specs=pl.BlockSpec((blk, d), lambda i: (i, 0)),
        out_shape=jax.ShapeDtypeStruct((n, d), F32),
    )(agg, den, x)


def kernel(x, edge_index, edge_h, edge_qrh, Wm, Wq, Wk):
    n, d = x.shape
    e = edge_index.shape[1]
    de = edge_h.shape[1]
    temp = float(d) ** 0.5

    import math
    chunk = math.lcm(NUM_SUBCORES * W_SC * 4,
                     NUM_CORES * NUM_SUBCORES * W_EDGES * 4)
    e_pad = ((e + chunk - 1) // chunk) * chunk
    pad = e_pad - e

    src = jnp.pad(edge_index[0], (0, pad))
    dst = jnp.pad(edge_index[1], (0, pad))

    wm1, wm2 = Wm[:d], Wm[d:]
    wk1, wk2 = Wk[:d], Wk[d:]
    wqt = jnp.transpose(Wq)

    xw2, xb = _node_kernel(x, wm1, wk1, wqt, temp)
    ew2 = _ew_kernel(edge_h, wm2, e, e_pad)
    xbs = _xbs_kernel(xb, src, e_pad)
    s = _s_kernel(edge_h, edge_qrh, xbs, wk2, wqt, temp, e, e_pad)

    xw2f = jnp.reshape(xw2, (2 * n, HALF))
    ew2f = jnp.reshape(ew2, (2 * e_pad, HALF))
    sf = jnp.reshape(s, (e_pad,))

    n_pad = ((n + 16 * 8 - 1) // (16 * 8)) * (16 * 8)
    agg, den = _scatter_kernel(xw2f, ew2f, src, dst, sf, n, n_pad, e_pad)
    return _final_kernel(agg, jnp.reshape(den, (n_pad, 1)), x)
